# baseline (speedup 1.0000x reference)
import functools

import jax
import jax.numpy as jnp
from jax import lax
from jax.experimental import pallas as pl
from jax.experimental.pallas import tpu as pltpu
from jax.experimental.pallas import tpu_sc as plsc

H = 128
N_NODES = 10000
N_EDGES = 320000
N_FRAMES = 5

NC = 2
NS = 16
NW = NC * NS

GW = 128
CHUNK = 128
NCHUNKS = N_EDGES // CHUNK

CHUNK_G = 80
E_PAD = 327680
NCHUNKS_G = E_PAD // CHUNK_G

AGG_PAD = 10240
ROWS_PER_TILE = AGG_PAD // NS
ZROWS = 64

EDGE_BLK = 2000


def _premix_body(h_ref, g_ref, ws_ref, wd_ref, be1_ref, a_ref, b_ref):
    h = h_ref[...]
    g = g_ref[...]
    dn = (((1,), (1,)), ((), ()))
    hs = lax.dot_general(h, ws_ref[...], dn, preferred_element_type=jnp.float32)
    hd = (
        lax.dot_general(h, wd_ref[...], dn, preferred_element_type=jnp.float32)
        + be1_ref[...]
    )
    a_ref[...] = jnp.concatenate([hs, g], axis=1)
    b_ref[...] = jnp.concatenate([hd, g], axis=1)


def _premix(h, g, ws, wd, be1):
    return pl.pallas_call(
        _premix_body,
        out_shape=(
            jax.ShapeDtypeStruct((N_NODES, H + GW), jnp.float32),
            jax.ShapeDtypeStruct((N_NODES, H + GW), jnp.float32),
        ),
    )(h, g, ws, wd, be1)


def _gather_body(a_hbm, b_hbm, src_hbm, dst_hbm,
                 oa, ob,
                 is0, id0, is1, id1, ba0, bb0, ba1, bb1,
                 s0, s1, s2, s3, s4, s5, s6, s7):
    wid = lax.axis_index("s") * NC + lax.axis_index("c")
    nk2 = NCHUNKS_G // (NW * 2)

    def body(t, _):
        b0 = (wid + (2 * t) * NW) * CHUNK_G
        b1 = (wid + (2 * t + 1) * NW) * CHUNK_G
        pltpu.sync_copy(src_hbm.at[pl.ds(b0, CHUNK_G)], is0)
        pltpu.sync_copy(dst_hbm.at[pl.ds(b0, CHUNK_G)], id0)
        c0 = pltpu.async_copy(a_hbm.at[is0], ba0, s0)
        c1 = pltpu.async_copy(b_hbm.at[id0], bb0, s1)
        pltpu.sync_copy(src_hbm.at[pl.ds(b1, CHUNK_G)], is1)
        pltpu.sync_copy(dst_hbm.at[pl.ds(b1, CHUNK_G)], id1)
        c2 = pltpu.async_copy(a_hbm.at[is1], ba1, s2)
        c3 = pltpu.async_copy(b_hbm.at[id1], bb1, s3)
        c0.wait()
        c1.wait()
        w0 = pltpu.async_copy(ba0, oa.at[pl.ds(b0, CHUNK_G)], s4)
        w1 = pltpu.async_copy(bb0, ob.at[pl.ds(b0, CHUNK_G)], s5)
        c2.wait()
        c3.wait()
        w2 = pltpu.async_copy(ba1, oa.at[pl.ds(b1, CHUNK_G)], s6)
        w3 = pltpu.async_copy(bb1, ob.at[pl.ds(b1, CHUNK_G)], s7)
        w0.wait()
        w1.wait()
        w2.wait()
        w3.wait()
        return 0

    lax.fori_loop(0, nk2, body, 0)


def _sc_gather(a, b, src, dst):
    mesh = plsc.VectorSubcoreMesh(core_axis_name="c", subcore_axis_name="s")
    f = pl.kernel(
        _gather_body,
        out_type=(
            jax.ShapeDtypeStruct((E_PAD, H + GW), jnp.float32),
            jax.ShapeDtypeStruct((E_PAD, H + GW), jnp.float32),
        ),
        mesh=mesh,
        scratch_types=[
            pltpu.VMEM((CHUNK_G,), jnp.int32),
            pltpu.VMEM((CHUNK_G,), jnp.int32),
            pltpu.VMEM((CHUNK_G,), jnp.int32),
            pltpu.VMEM((CHUNK_G,), jnp.int32),
            pltpu.VMEM((CHUNK_G, H + GW), jnp.float32),
            pltpu.VMEM((CHUNK_G, H + GW), jnp.float32),
            pltpu.VMEM((CHUNK_G, H + GW), jnp.float32),
            pltpu.VMEM((CHUNK_G, H + GW), jnp.float32),
            pltpu.SemaphoreType.DMA,
            pltpu.SemaphoreType.DMA,
            pltpu.SemaphoreType.DMA,
            pltpu.SemaphoreType.DMA,
            pltpu.SemaphoreType.DMA,
            pltpu.SemaphoreType.DMA,
            pltpu.SemaphoreType.DMA,
            pltpu.SemaphoreType.DMA,
        ],
    )
    return f(a, b, src, dst)


def _edge_body(ba_ref, bb_ref, wg_ref, we2_ref, be2_ref, out_ref):
    a = ba_ref[...]
    b = bb_ref[...]
    gs = a[:, H:H + 18]
    gd = b[:, H:H + 18]
    rel = gs[:, 0:3] - gd[:, 0:3]
    dist2 = jnp.sum(rel * rel, axis=1, keepdims=True)
    dist = jnp.maximum(jnp.sqrt(dist2), 1e-8)
    inv = 1.0 / dist
    r0 = rel[:, 0:1]
    r1 = rel[:, 1:2]
    r2 = rel[:, 2:3]
    vps = (gs[:, 3:8] * r0 + gs[:, 8:13] * r1 + gs[:, 13:18] * r2) * inv
    vpd = (gd[:, 3:8] * r0 + gd[:, 8:13] * r1 + gd[:, 13:18] * r2) * inv
    zpad = jnp.zeros((dist2.shape[0], 5), jnp.float32)
    feat = jnp.concatenate([dist2, vps, vpd, zpad], axis=1)
    dn = (((1,), (1,)), ((), ()))
    pre = (
        a[:, 0:H]
        + b[:, 0:H]
        + lax.dot_general(feat, wg_ref[...], dn, preferred_element_type=jnp.float32)
    )
    m = jax.nn.silu(pre)
    mm = lax.dot_general(m, we2_ref[...], dn, preferred_element_type=jnp.float32) + be2_ref[...]
    out_ref[...] = jax.nn.silu(mm)


def _edge_mlp(ba, bb, wg16, we2, be2):
    nblk = N_EDGES // EDGE_BLK
    return pl.pallas_call(
        _edge_body,
        grid=(nblk,),
        in_specs=[
            pl.BlockSpec((EDGE_BLK, H + GW), lambda i: (i, 0)),
            pl.BlockSpec((EDGE_BLK, H + GW), lambda i: (i, 0)),
            pl.BlockSpec((H, 16), lambda i: (0, 0)),
            pl.BlockSpec((H, H), lambda i: (0, 0)),
            pl.BlockSpec((1, H), lambda i: (0, 0)),
        ],
        out_specs=pl.BlockSpec((EDGE_BLK, H), lambda i: (i, 0)),
        out_shape=jax.ShapeDtypeStruct((N_EDGES, H), jnp.float32),
    )(ba, bb, wg16, we2, be2)


def _scatter_body(m_hbm, dst_hbm, out_hbm, agg_sh, idx_d, buf, zbuf):
    cid = lax.axis_index("c")
    sid = lax.axis_index("s")
    wid = sid * NC + cid

    def zr(i, _):
        r = i // 8
        c = (i % 8) * 16
        zbuf[r, pl.ds(c, 16)] = jnp.zeros((16,), jnp.float32)
        return 0

    lax.fori_loop(0, ZROWS * 8, zr, 0)
    tb = sid * ROWS_PER_TILE

    def zcopy(i, _):
        pltpu.sync_copy(zbuf, agg_sh.at[pl.ds(tb + i * ZROWS, ZROWS)])
        return 0

    lax.fori_loop(0, ROWS_PER_TILE // ZROWS, zcopy, 0)
    plsc.subcore_barrier()

    nk = (NCHUNKS - wid + NW - 1) // NW

    def body(k, _):
        j = wid + k * NW
        base = j * CHUNK
        pltpu.sync_copy(dst_hbm.at[pl.ds(base, CHUNK)], idx_d)
        pltpu.sync_copy(m_hbm.at[pl.ds(base, CHUNK)], buf)
        pltpu.sync_copy(buf, agg_sh.at[idx_d], add=True)
        return 0

    lax.fori_loop(0, nk, body, 0)
    plsc.subcore_barrier()

    ob = cid * AGG_PAD + tb

    def dump(i, _):
        pltpu.sync_copy(agg_sh.at[pl.ds(tb + i * ZROWS, ZROWS)], zbuf)
        pltpu.sync_copy(zbuf, out_hbm.at[pl.ds(ob + i * ZROWS, ZROWS)])
        return 0

    lax.fori_loop(0, ROWS_PER_TILE // ZROWS, dump, 0)


def _sc_scatter(m_ij, dst):
    mesh = plsc.VectorSubcoreMesh(core_axis_name="c", subcore_axis_name="s")
    f = pl.kernel(
        _scatter_body,
        out_type=jax.ShapeDtypeStruct((NC * AGG_PAD, H), jnp.float32),
        mesh=mesh,
        scratch_types=[
            pltpu.VMEM_SHARED((AGG_PAD, H), jnp.float32),
            pltpu.VMEM((CHUNK,), jnp.int32),
            pltpu.VMEM((CHUNK, H), jnp.float32),
            pltpu.VMEM((ZROWS, H), jnp.float32),
        ],
    )
    return f(m_ij, dst)


def _node_body(h_ref, p_ref, w1a_ref, w1b_ref, bh1_ref, w2_ref, bh2_ref,
               g_ref, b_ref, out_ref):
    h = h_ref[...]
    agg = p_ref[0:N_NODES, :] + p_ref[AGG_PAD:AGG_PAD + N_NODES, :]
    dn = (((1,), (1,)), ((), ()))
    t = (
        lax.dot_general(h, w1a_ref[...], dn, preferred_element_type=jnp.float32)
        + lax.dot_general(agg, w1b_ref[...], dn, preferred_element_type=jnp.float32)
        + bh1_ref[...]
    )
    t = jax.nn.silu(t)
    dh = lax.dot_general(t, w2_ref[...], dn, preferred_element_type=jnp.float32) + bh2_ref[...]
    h2 = h + dh
    mean = jnp.mean(h2, axis=1, keepdims=True)
    cen = h2 - mean
    var = jnp.mean(cen * cen, axis=1, keepdims=True)
    out_ref[...] = cen * lax.rsqrt(var + 1e-5) * g_ref[...] + b_ref[...]


def _node_mlp(h, partials, w1a, w1b, bh1, w2, bh2, ln_g, ln_b):
    return pl.pallas_call(
        _node_body,
        out_shape=jax.ShapeDtypeStruct((N_NODES, H), jnp.float32),
    )(h, partials, w1a, w1b, bh1, w2, bh2, ln_g, ln_b)


@jax.jit
def kernel(h, x, vel_all, edge_index, We1, be1, We2, be2, Wh1, bh1, Wh2, bh2, ln_g, ln_b):
    src = edge_index[0]
    dst = edge_index[1]

    ws = We1[:, :H]
    wd = We1[:, H:2 * H]
    wg16 = jnp.zeros((H, 16), jnp.float32).at[:, :11].set(We1[:, 2 * H:])

    g = jnp.concatenate(
        [x, vel_all[:, :, 0], vel_all[:, :, 1], vel_all[:, :, 2],
         jnp.zeros((N_NODES, GW - 18), jnp.float32)],
        axis=1,
    )

    a, b = _premix(h, g, ws, wd, be1.reshape(1, H))
    zpad_e = jnp.zeros((E_PAD - N_EDGES,), jnp.int32)
    ba, bb = _sc_gather(a, b,
                        jnp.concatenate([src, zpad_e]),
                        jnp.concatenate([dst, zpad_e]))
    m_ij = _edge_mlp(ba, bb, wg16, We2, be2.reshape(1, H))
    partials = _sc_scatter(m_ij, dst)
    h_norm = _node_mlp(
        h, partials,
        Wh1[:, :H], Wh1[:, H:], bh1.reshape(1, H),
        Wh2, bh2.reshape(1, H), ln_g.reshape(1, H), ln_b.reshape(1, H),
    )
    return (h_norm, x, m_ij)


# speedup vs baseline: 1.3800x; 1.3800x over previous
import functools

import jax
import jax.numpy as jnp
from jax import lax
from jax.experimental import pallas as pl
from jax.experimental.pallas import tpu as pltpu
from jax.experimental.pallas import tpu_sc as plsc

H = 128
N_NODES = 10000
N_EDGES = 320000
N_FRAMES = 5

NC = 2
NS = 16
NW = NC * NS

CHUNK = 128
NCHUNKS = N_EDGES // CHUNK

AGG_PAD = 10240
ROWS_PER_TILE = AGG_PAD // NS
ZROWS = 64

EDGE_BLK = 2000


def _pack_lo_hi(lo_f32, hi_f32):
    lo = lax.bitcast_convert_type(lo_f32, jnp.uint32)
    hi = lax.bitcast_convert_type(hi_f32, jnp.uint32)
    lo = (lo + jnp.uint32(0x8000)) >> jnp.uint32(16)
    hi = (hi + jnp.uint32(0x8000)) & jnp.uint32(0xFFFF0000)
    return lax.bitcast_convert_type(lo | hi, jnp.int32)


def _premix_body(h_ref, g_ref, ws_ref, wd_ref, be1_ref, a_ref, b_ref):
    h = h_ref[...]
    g = g_ref[...]
    dn = (((1,), (1,)), ((), ()))
    hs = lax.dot_general(h, ws_ref[...], dn, preferred_element_type=jnp.float32)
    hd = (
        lax.dot_general(h, wd_ref[...], dn, preferred_element_type=jnp.float32)
        + be1_ref[...]
    )
    a_ref[...] = _pack_lo_hi(hs, g)
    b_ref[...] = _pack_lo_hi(hd, g)


def _premix(h, g, ws, wd, be1):
    return pl.pallas_call(
        _premix_body,
        out_shape=(
            jax.ShapeDtypeStruct((N_NODES, H), jnp.int32),
            jax.ShapeDtypeStruct((N_NODES, H), jnp.int32),
        ),
    )(h, g, ws, wd, be1)


def _gather_body(a_hbm, b_hbm, src_hbm, dst_hbm,
                 oa, ob,
                 idx_s, idx_d, ba, bb, s0, s1):
    wid = lax.axis_index("s") * NC + lax.axis_index("c")
    nk = (NCHUNKS - wid + NW - 1) // NW

    def body(k, _):
        j = wid + k * NW
        base = j * CHUNK
        pltpu.sync_copy(src_hbm.at[pl.ds(base, CHUNK)], idx_s)
        pltpu.sync_copy(dst_hbm.at[pl.ds(base, CHUNK)], idx_d)
        c0 = pltpu.async_copy(a_hbm.at[idx_s], ba, s0)
        c1 = pltpu.async_copy(b_hbm.at[idx_d], bb, s1)
        c0.wait()
        c1.wait()
        pltpu.sync_copy(ba, oa.at[pl.ds(base, CHUNK)])
        pltpu.sync_copy(bb, ob.at[pl.ds(base, CHUNK)])
        return 0

    lax.fori_loop(0, nk, body, 0)


def _sc_gather(a, b, src, dst):
    mesh = plsc.VectorSubcoreMesh(core_axis_name="c", subcore_axis_name="s")
    f = pl.kernel(
        _gather_body,
        out_type=(
            jax.ShapeDtypeStruct((N_EDGES, H), jnp.int32),
            jax.ShapeDtypeStruct((N_EDGES, H), jnp.int32),
        ),
        mesh=mesh,
        scratch_types=[
            pltpu.VMEM((CHUNK,), jnp.int32),
            pltpu.VMEM((CHUNK,), jnp.int32),
            pltpu.VMEM((CHUNK, H), jnp.int32),
            pltpu.VMEM((CHUNK, H), jnp.int32),
            pltpu.SemaphoreType.DMA,
            pltpu.SemaphoreType.DMA,
        ],
    )
    return f(a, b, src, dst)


def _unpack_lo_hi(w):
    lo = lax.bitcast_convert_type(w << 16, jnp.float32)
    hi = lax.bitcast_convert_type(w & jnp.int32(-65536), jnp.float32)
    return lo, hi


def _edge_body(ba_ref, bb_ref, wg_ref, we2_ref, be2_ref, out_ref):
    a, ga = _unpack_lo_hi(ba_ref[...])
    b, gb = _unpack_lo_hi(bb_ref[...])
    gs = ga[:, 0:18]
    gd = gb[:, 0:18]
    rel = gs[:, 0:3] - gd[:, 0:3]
    dist2 = jnp.sum(rel * rel, axis=1, keepdims=True)
    dist = jnp.maximum(jnp.sqrt(dist2), 1e-8)
    inv = 1.0 / dist
    r0 = rel[:, 0:1]
    r1 = rel[:, 1:2]
    r2 = rel[:, 2:3]
    vps = (gs[:, 3:8] * r0 + gs[:, 8:13] * r1 + gs[:, 13:18] * r2) * inv
    vpd = (gd[:, 3:8] * r0 + gd[:, 8:13] * r1 + gd[:, 13:18] * r2) * inv
    zpad = jnp.zeros((dist2.shape[0], 5), jnp.float32)
    feat = jnp.concatenate([dist2, vps, vpd, zpad], axis=1)
    dn = (((1,), (1,)), ((), ()))
    pre = (
        a
        + b
        + lax.dot_general(feat, wg_ref[...], dn, preferred_element_type=jnp.float32)
    )
    m = jax.nn.silu(pre)
    mm = lax.dot_general(m, we2_ref[...], dn, preferred_element_type=jnp.float32) + be2_ref[...]
    out_ref[...] = jax.nn.silu(mm)


def _edge_mlp(ba, bb, wg16, we2, be2):
    nblk = N_EDGES // EDGE_BLK
    return pl.pallas_call(
        _edge_body,
        grid=(nblk,),
        in_specs=[
            pl.BlockSpec((EDGE_BLK, H), lambda i: (i, 0)),
            pl.BlockSpec((EDGE_BLK, H), lambda i: (i, 0)),
            pl.BlockSpec((H, 16), lambda i: (0, 0)),
            pl.BlockSpec((H, H), lambda i: (0, 0)),
            pl.BlockSpec((1, H), lambda i: (0, 0)),
        ],
        out_specs=pl.BlockSpec((EDGE_BLK, H), lambda i: (i, 0)),
        out_shape=jax.ShapeDtypeStruct((N_EDGES, H), jnp.float32),
    )(ba, bb, wg16, we2, be2)


def _scatter_body(m_hbm, dst_hbm, out_hbm, agg_sh, idx_d, buf, zbuf):
    cid = lax.axis_index("c")
    sid = lax.axis_index("s")
    wid = sid * NC + cid

    def zr(i, _):
        r = i // 8
        c = (i % 8) * 16
        zbuf[r, pl.ds(c, 16)] = jnp.zeros((16,), jnp.float32)
        return 0

    lax.fori_loop(0, ZROWS * 8, zr, 0)
    tb = sid * ROWS_PER_TILE

    def zcopy(i, _):
        pltpu.sync_copy(zbuf, agg_sh.at[pl.ds(tb + i * ZROWS, ZROWS)])
        return 0

    lax.fori_loop(0, ROWS_PER_TILE // ZROWS, zcopy, 0)
    plsc.subcore_barrier()

    nk = (NCHUNKS - wid + NW - 1) // NW

    def body(k, _):
        j = wid + k * NW
        base = j * CHUNK
        pltpu.sync_copy(dst_hbm.at[pl.ds(base, CHUNK)], idx_d)
        pltpu.sync_copy(m_hbm.at[pl.ds(base, CHUNK)], buf)
        pltpu.sync_copy(buf, agg_sh.at[idx_d], add=True)
        return 0

    lax.fori_loop(0, nk, body, 0)
    plsc.subcore_barrier()

    ob = cid * AGG_PAD + tb

    def dump(i, _):
        pltpu.sync_copy(agg_sh.at[pl.ds(tb + i * ZROWS, ZROWS)], zbuf)
        pltpu.sync_copy(zbuf, out_hbm.at[pl.ds(ob + i * ZROWS, ZROWS)])
        return 0

    lax.fori_loop(0, ROWS_PER_TILE // ZROWS, dump, 0)


def _sc_scatter(m_ij, dst):
    mesh = plsc.VectorSubcoreMesh(core_axis_name="c", subcore_axis_name="s")
    f = pl.kernel(
        _scatter_body,
        out_type=jax.ShapeDtypeStruct((NC * AGG_PAD, H), jnp.float32),
        mesh=mesh,
        scratch_types=[
            pltpu.VMEM_SHARED((AGG_PAD, H), jnp.float32),
            pltpu.VMEM((CHUNK,), jnp.int32),
            pltpu.VMEM((CHUNK, H), jnp.float32),
            pltpu.VMEM((ZROWS, H), jnp.float32),
        ],
    )
    return f(m_ij, dst)


def _node_body(h_ref, p_ref, w1a_ref, w1b_ref, bh1_ref, w2_ref, bh2_ref,
               g_ref, b_ref, out_ref):
    h = h_ref[...]
    agg = p_ref[0:N_NODES, :] + p_ref[AGG_PAD:AGG_PAD + N_NODES, :]
    dn = (((1,), (1,)), ((), ()))
    t = (
        lax.dot_general(h, w1a_ref[...], dn, preferred_element_type=jnp.float32)
        + lax.dot_general(agg, w1b_ref[...], dn, preferred_element_type=jnp.float32)
        + bh1_ref[...]
    )
    t = jax.nn.silu(t)
    dh = lax.dot_general(t, w2_ref[...], dn, preferred_element_type=jnp.float32) + bh2_ref[...]
    h2 = h + dh
    mean = jnp.mean(h2, axis=1, keepdims=True)
    cen = h2 - mean
    var = jnp.mean(cen * cen, axis=1, keepdims=True)
    out_ref[...] = cen * lax.rsqrt(var + 1e-5) * g_ref[...] + b_ref[...]


def _node_mlp(h, partials, w1a, w1b, bh1, w2, bh2, ln_g, ln_b):
    return pl.pallas_call(
        _node_body,
        out_shape=jax.ShapeDtypeStruct((N_NODES, H), jnp.float32),
    )(h, partials, w1a, w1b, bh1, w2, bh2, ln_g, ln_b)


@jax.jit
def kernel(h, x, vel_all, edge_index, We1, be1, We2, be2, Wh1, bh1, Wh2, bh2, ln_g, ln_b):
    src = edge_index[0]
    dst = edge_index[1]

    ws = We1[:, :H]
    wd = We1[:, H:2 * H]
    wg16 = jnp.zeros((H, 16), jnp.float32).at[:, :11].set(We1[:, 2 * H:])

    g = jnp.concatenate(
        [x, vel_all[:, :, 0], vel_all[:, :, 1], vel_all[:, :, 2],
         jnp.zeros((N_NODES, H - 18), jnp.float32)],
        axis=1,
    )

    a, b = _premix(h, g, ws, wd, be1.reshape(1, H))
    ba, bb = _sc_gather(a, b, src, dst)
    m_ij = _edge_mlp(ba, bb, wg16, We2, be2.reshape(1, H))
    partials = _sc_scatter(m_ij, dst)
    h_norm = _node_mlp(
        h, partials,
        Wh1[:, :H], Wh1[:, H:], bh1.reshape(1, H),
        Wh2, bh2.reshape(1, H), ln_g.reshape(1, H), ln_b.reshape(1, H),
    )
    return (h_norm, x, m_ij)
